# baseline (device time: 196257 ns/iter reference)
import jax
import jax.numpy as jnp
from jax import lax
from jax.experimental import pallas as pl
from jax.experimental.pallas import tpu as pltpu

S = 2048
K = 4096
N = 8192
HALF = 1024
CW = 512
NC = N // CW
NH = NC // 2
STEPS = 3 * NH


def _c16(t, s):
    mx = s[0]
    mine = NH * mx
    other = NH * (1 - mx)
    return jnp.where(t < NH, mine + t,
                     jnp.where(t < 2 * NH, mine + t - NH, other + t - 2 * NH))


def _fused(scalars, A, Wo):
    def body(s_ref, a_ref, w_ref, o_ref, send_buf, recv_buf,
             y_send_sems, fwd_send_sems, recv_sems):
        t = pl.program_id(0)
        mx = s_ref[0]
        my = s_ref[1]
        y_peer = (mx, 1 - my)
        x_peer = (1 - mx, my)

        @pl.when(t == 0)
        def _():
            barrier_sem = pltpu.get_barrier_semaphore()
            for nbr in (y_peer, x_peer):
                pl.semaphore_signal(barrier_sem, inc=1, device_id=nbr,
                                    device_id_type=pl.DeviceIdType.MESH)
            pl.semaphore_wait(barrier_sem, 2)

        chunk = lax.dot_general(
            a_ref[...], w_ref[...], (((1,), (0,)), ((), ())),
            preferred_element_type=jnp.float32)

        send_slot = NH * mx + t
        recv_slot = jnp.where(t < 2 * NH, NH * mx + t - NH,
                              NH * (1 - mx) + t - 2 * NH)

        @pl.when(t < NH)
        def _():
            send_buf[t] = chunk.astype(jnp.bfloat16)
            rdma = pltpu.make_async_remote_copy(
                src_ref=send_buf.at[t],
                dst_ref=recv_buf.at[send_slot],
                send_sem=y_send_sems.at[t],
                recv_sem=recv_sems.at[send_slot],
                device_id=y_peer,
                device_id_type=pl.DeviceIdType.MESH,
            )
            rdma.start()

        @pl.when(t >= NH)
        def _():
            recv_desc = pltpu.make_async_remote_copy(
                src_ref=recv_buf.at[recv_slot],
                dst_ref=recv_buf.at[recv_slot],
                send_sem=y_send_sems.at[0],
                recv_sem=recv_sems.at[recv_slot],
                device_id=y_peer,
                device_id_type=pl.DeviceIdType.MESH,
            )
            recv_desc.wait_recv()

            @pl.when(t < 2 * NH)
            def _():
                fwd = pltpu.make_async_remote_copy(
                    src_ref=recv_buf.at[recv_slot],
                    dst_ref=recv_buf.at[recv_slot],
                    send_sem=fwd_send_sems.at[t - NH],
                    recv_sem=recv_sems.at[recv_slot],
                    device_id=x_peer,
                    device_id_type=pl.DeviceIdType.MESH,
                )
                fwd.start()

            o_ref[0] = chunk + recv_buf[recv_slot].astype(jnp.float32)

        @pl.when(t == STEPS - 1)
        def _():
            for c in range(NH):
                pltpu.make_async_remote_copy(
                    src_ref=send_buf.at[c],
                    dst_ref=recv_buf.at[c],
                    send_sem=y_send_sems.at[c],
                    recv_sem=recv_sems.at[c],
                    device_id=y_peer,
                    device_id_type=pl.DeviceIdType.MESH,
                ).wait_send()
                pltpu.make_async_remote_copy(
                    src_ref=recv_buf.at[NH * mx + c],
                    dst_ref=recv_buf.at[NH * mx + c],
                    send_sem=fwd_send_sems.at[c],
                    recv_sem=recv_sems.at[c],
                    device_id=x_peer,
                    device_id_type=pl.DeviceIdType.MESH,
                ).wait_send()

    grid_spec = pltpu.PrefetchScalarGridSpec(
        num_scalar_prefetch=1,
        grid=(STEPS,),
        in_specs=[
            pl.BlockSpec(
                (HALF, K),
                lambda t, s: (jnp.where(t < NH, 1 - s[1], s[1]), 0)),
            pl.BlockSpec((K, CW), lambda t, s: (0, _c16(t, s))),
        ],
        out_specs=pl.BlockSpec(
            (1, HALF, CW),
            lambda t, s: (0, 0, jnp.where(t < NH, NH * s[0], _c16(t, s)))),
        scratch_shapes=[
            pltpu.VMEM((NH, HALF, CW), jnp.bfloat16),
            pltpu.VMEM((NC, HALF, CW), jnp.bfloat16),
            pltpu.SemaphoreType.DMA((NH,)),
            pltpu.SemaphoreType.DMA((NH,)),
            pltpu.SemaphoreType.DMA((NC,)),
        ],
    )
    return pl.pallas_call(
        body,
        grid_spec=grid_spec,
        out_shape=jax.ShapeDtypeStruct((1, HALF, N), jnp.float32),
        compiler_params=pltpu.CompilerParams(
            collective_id=0,
            dimension_semantics=("arbitrary",),
            vmem_limit_bytes=100 * 1024 * 1024),
    )(scalars, A, Wo)


def kernel(O, Wo):
    mx = lax.axis_index("x")
    my = lax.axis_index("y")
    A = O.reshape(S, K).astype(jnp.bfloat16)
    scalars = jnp.stack([mx, my]).astype(jnp.int32)
    return _fused(scalars, A, Wo)


# device time: 196129 ns/iter; 1.0007x vs baseline; 1.0007x over previous
import jax
import jax.numpy as jnp
from jax import lax
from jax.experimental import pallas as pl
from jax.experimental.pallas import tpu as pltpu

S = 2048
K = 4096
N = 8192
HALF = 1024
CW = 512
NC = N // CW
NH = NC // 2
STEPS = 3 * NH


def _c16(t, s):
    mx = s[0]
    mine = NH * mx
    other = NH * (1 - mx)
    return jnp.where(t < NH, mine + t,
                     jnp.where(t < 2 * NH, mine + t - NH, other + t - 2 * NH))


def _fused(scalars, A, Wo):
    def body(s_ref, a_ref, w_ref, o_ref, send_buf, recv_buf,
             y_send_sems, fwd_send_sems, recv_sems):
        t = pl.program_id(0)
        mx = s_ref[0]
        my = s_ref[1]
        y_peer = (mx, 1 - my)
        x_peer = (1 - mx, my)

        @pl.when(t == 0)
        def _():
            barrier_sem = pltpu.get_barrier_semaphore()
            for nbr in (y_peer, x_peer):
                pl.semaphore_signal(barrier_sem, inc=1, device_id=nbr,
                                    device_id_type=pl.DeviceIdType.MESH)
            pl.semaphore_wait(barrier_sem, 2)

        a = a_ref[...]
        wb0 = w_ref[:, : CW // 2].astype(jnp.bfloat16)
        wb1 = w_ref[:, CW // 2 :].astype(jnp.bfloat16)
        c0 = jnp.dot(a, wb0, preferred_element_type=jnp.float32)
        c1 = jnp.dot(a, wb1, preferred_element_type=jnp.float32)
        chunk = jnp.concatenate([c0, c1], axis=1)

        send_slot = NH * mx + t
        recv_slot = jnp.where(t < 2 * NH, NH * mx + t - NH,
                              NH * (1 - mx) + t - 2 * NH)

        @pl.when(t < NH)
        def _():
            send_buf[t] = chunk.astype(jnp.bfloat16)
            rdma = pltpu.make_async_remote_copy(
                src_ref=send_buf.at[t],
                dst_ref=recv_buf.at[send_slot],
                send_sem=y_send_sems.at[t],
                recv_sem=recv_sems.at[send_slot],
                device_id=y_peer,
                device_id_type=pl.DeviceIdType.MESH,
            )
            rdma.start()

        @pl.when(t >= NH)
        def _():
            recv_desc = pltpu.make_async_remote_copy(
                src_ref=recv_buf.at[recv_slot],
                dst_ref=recv_buf.at[recv_slot],
                send_sem=y_send_sems.at[0],
                recv_sem=recv_sems.at[recv_slot],
                device_id=y_peer,
                device_id_type=pl.DeviceIdType.MESH,
            )
            recv_desc.wait_recv()

            @pl.when(t < 2 * NH)
            def _():
                fwd = pltpu.make_async_remote_copy(
                    src_ref=recv_buf.at[recv_slot],
                    dst_ref=recv_buf.at[recv_slot],
                    send_sem=fwd_send_sems.at[t - NH],
                    recv_sem=recv_sems.at[recv_slot],
                    device_id=x_peer,
                    device_id_type=pl.DeviceIdType.MESH,
                )
                fwd.start()

            o_ref[0] = chunk + recv_buf[recv_slot].astype(jnp.float32)

        @pl.when(t == STEPS - 1)
        def _():
            for c in range(NH):
                pltpu.make_async_remote_copy(
                    src_ref=send_buf.at[c],
                    dst_ref=recv_buf.at[c],
                    send_sem=y_send_sems.at[c],
                    recv_sem=recv_sems.at[c],
                    device_id=y_peer,
                    device_id_type=pl.DeviceIdType.MESH,
                ).wait_send()
                pltpu.make_async_remote_copy(
                    src_ref=recv_buf.at[NH * mx + c],
                    dst_ref=recv_buf.at[NH * mx + c],
                    send_sem=fwd_send_sems.at[c],
                    recv_sem=recv_sems.at[c],
                    device_id=x_peer,
                    device_id_type=pl.DeviceIdType.MESH,
                ).wait_send()

    grid_spec = pltpu.PrefetchScalarGridSpec(
        num_scalar_prefetch=1,
        grid=(STEPS,),
        in_specs=[
            pl.BlockSpec(
                (HALF, K),
                lambda t, s: (jnp.where(t < NH, 1 - s[1], s[1]), 0)),
            pl.BlockSpec((K, CW), lambda t, s: (0, _c16(t, s))),
        ],
        out_specs=pl.BlockSpec(
            (1, HALF, CW),
            lambda t, s: (0, 0, jnp.where(t < NH, NH * s[0], _c16(t, s)))),
        scratch_shapes=[
            pltpu.VMEM((NH, HALF, CW), jnp.bfloat16),
            pltpu.VMEM((NC, HALF, CW), jnp.bfloat16),
            pltpu.SemaphoreType.DMA((NH,)),
            pltpu.SemaphoreType.DMA((NH,)),
            pltpu.SemaphoreType.DMA((NC,)),
        ],
    )
    return pl.pallas_call(
        body,
        grid_spec=grid_spec,
        out_shape=jax.ShapeDtypeStruct((1, HALF, N), jnp.float32),
        compiler_params=pltpu.CompilerParams(
            collective_id=0,
            dimension_semantics=("arbitrary",),
            vmem_limit_bytes=100 * 1024 * 1024),
    )(scalars, A, Wo)


def kernel(O, Wo):
    mx = lax.axis_index("x")
    my = lax.axis_index("y")
    A = O.reshape(S, K).astype(jnp.bfloat16)
    scalars = jnp.stack([mx, my]).astype(jnp.int32)
    return _fused(scalars, A, Wo)
